# baseline (speedup 1.0000x reference)
import functools

import jax
import jax.numpy as jnp
from jax import lax
from jax.experimental import pallas as pl
from jax.experimental.pallas import tpu as pltpu
from jax.experimental.pallas import tpu_sc as plsc

_N = 1048576
_CELLS = 128
_NC, _NS, _L = 2, 16, 16
_NW = _NC * _NS
_PPW = _N // _NW
_C = 4096
_NCHUNK = _PPW // _C
_GP = _C // _L
_SROWS = 64 * 64 * 32

_mesh = plsc.VectorSubcoreMesh(core_axis_name="c", subcore_axis_name="s")


@functools.partial(
    pl.kernel,
    out_type=(
        jax.ShapeDtypeStruct((4 * _N,), jnp.float32),
        jax.ShapeDtypeStruct((_N,), jnp.float32),
        jax.ShapeDtypeStruct((2 * _SROWS, 8), jnp.float32),
    ),
    mesh=_mesh,
    compiler_params=pltpu.CompilerParams(
        needs_layout_passes=False, use_tc_tiling_on_sc=False),
    scratch_types=[
        pltpu.VMEM((_C,), jnp.float32),
        pltpu.VMEM((_C,), jnp.float32),
        pltpu.VMEM((_C,), jnp.float32),
        pltpu.VMEM((_C,), jnp.float32),
        pltpu.VMEM((_C,), jnp.int32),
        pltpu.VMEM((_C,), jnp.int32),
        pltpu.VMEM((_C,), jnp.int32),
        pltpu.VMEM((_C, 8), jnp.float32),
        pltpu.VMEM((4 * _C,), jnp.float32),
        pltpu.VMEM((_C,), jnp.float32),
        pltpu.VMEM((4096,), jnp.float32),
        pltpu.VMEM((4096,), jnp.float32),
        pltpu.VMEM((256, 8), jnp.float32),
        pltpu.VMEM((256, 8), jnp.float32),
        pltpu.SemaphoreType.DMA,
        pltpu.SemaphoreType.DMA,
        pltpu.SemaphoreType.DMA,
        pltpu.SemaphoreType.DMA,
        pltpu.SemaphoreType.DMA,
    ],
)
def _voxel_sc(xp_hbm, grid_hbm, rgb_hbm, den_hbm, scr_hbm,
              xv, yv, zv, cond_v, pos_v, sel_v, i0, vals_v, rgb_v, den_v,
              inb0, inb1, outb0, outb1, semg, semi0, semi1, semo0, semo1):
    scid = lax.axis_index("c")
    sid = lax.axis_index("s")
    wid = sid * _NC + scid
    lanes = lax.iota(jnp.int32, _L)
    shuf_static = ((lanes & 3) << 7) + ((lanes >> 3) << 1) + ((lanes >> 2) & 1)

    inbufs = (inb0, inb1)
    outbufs = (outb0, outb1)
    insems = (semi0, semi1)
    outsems = (semo0, semo1)

    def slab_copies(it):
        src_off = ((64 + sid * 4 + it // 8) * 128 + 64 + (it % 8) * 8) * 512
        dst_row = scid * _SROWS + ((sid * 4 + it // 8) * 64 + (it % 8) * 8) * 32
        inc = pltpu.make_async_copy(grid_hbm.at[pl.ds(src_off, 4096)],
                                    inbufs[it % 2], insems[it % 2])
        outc = pltpu.make_async_copy(outbufs[it % 2],
                                     scr_hbm.at[pl.ds(dst_row, 256)],
                                     outsems[it % 2])
        return inc, outc

    slab_copies(0)[0].start()
    for it in range(32):
        inc, outc = slab_copies(it)
        if it + 1 < 32:
            slab_copies(it + 1)[0].start()
        inc.wait()
        if it >= 2:
            slab_copies(it - 2)[1].wait()
        iv = inbufs[it % 2]
        ov = outbufs[it % 2]

        def shuf(g, _):
            base = ((g >> 4) << 9) + ((g & 15) << 2) + 64
            vals = plsc.load_gather(iv, [base + shuf_static])
            plsc.store_scatter(ov, [(g * 2) + (lanes >> 3), lanes & 7], vals)
            return 0

        lax.fori_loop(0, 128, shuf, 0)
        outc.start()
    slab_copies(30)[1].wait()
    slab_copies(31)[1].wait()
    plsc.subcore_barrier()

    def to_cell(v):
        i = (v * jnp.float32(_CELLS) + jnp.float32(_CELLS // 2)).astype(jnp.int32)
        return jnp.clip(i, 0, _CELLS - 1)

    zeros16 = jnp.zeros((_L,), jnp.int32)

    def zinit(i, _):
        i0[pl.ds(i * _L, _L)] = zeros16
        return 0

    lax.fori_loop(0, _GP, zinit, 0)

    ch_splats = [jnp.full((_L,), c, jnp.int32) for c in range(4)]

    def chunk_body(ci, _):
        base = wid * _PPW + ci * _C
        pltpu.sync_copy(xp_hbm.at[pl.ds(base, _C)], xv)
        pltpu.sync_copy(xp_hbm.at[pl.ds(_N + base, _C)], yv)
        pltpu.sync_copy(xp_hbm.at[pl.ds(2 * _N + base, _C)], zv)

        def pass1(g, off):
            s = g * _L
            x = xv[pl.ds(s, _L)]
            y = yv[pl.ds(s, _L)]
            z = zv[pl.ds(s, _L)]
            half = jnp.float32(0.5)
            cond = ((jnp.abs(x) < half) & (jnp.abs(y) < half)
                    & (jnp.abs(z) < half))
            maski = cond.astype(jnp.int32)
            scell = (((to_cell(x) - 64) * 64 + (to_cell(y) - 64)) * 64
                     + (to_cell(z) - 64))
            cond_v[pl.ds(s, _L)] = jnp.where(cond, 1.0, 0.0).astype(jnp.float32)
            sel_v[pl.ds(s, _L)] = (scell & 1) << 2
            pos_v[pl.ds(s, _L)] = off + plsc.cumsum(maski) - 1
            plsc.store_compressed(i0.at[pl.ds(off, _L)],
                                  (scell >> 1) + scid * _SROWS, mask=cond)
            return off + jnp.sum(maski)

        cnt = lax.fori_loop(0, _GP, pass1, jnp.int32(0))
        ndma = (cnt + 127) >> 7

        def fire(j, _):
            pltpu.make_async_copy(
                scr_hbm.at[i0.at[pl.ds(j * 128, 128)]],
                vals_v.at[pl.ds(j * 128, 128)], semg).start()
            return 0

        lax.fori_loop(0, ndma, fire, 0)

        def drain(j, _):
            pltpu.make_async_copy(
                scr_hbm.at[i0.at[pl.ds(j * 128, 128)]],
                vals_v.at[pl.ds(j * 128, 128)], semg).wait()
            return 0

        lax.fori_loop(0, ndma, drain, 0)

        def pass2(g, _):
            s = g * _L
            condf = cond_v[pl.ds(s, _L)]
            mask = condf > jnp.float32(0.5)
            pos = pos_v[pl.ds(s, _L)]
            sel = sel_v[pl.ds(s, _L)]
            obase = (g >> 3) * 512 + (g & 7) * _L
            one = jnp.float32(1.0)
            for c in range(3):
                v = plsc.load_gather(vals_v, [pos, sel + ch_splats[c]],
                                     mask=mask)
                v = jnp.where(mask, v, jnp.float32(0.0))
                rgb_v[pl.ds(obase + c * 128, _L)] = one / (one + jnp.exp(-v))
            d = plsc.load_gather(vals_v, [pos, sel + ch_splats[3]], mask=mask)
            d = jnp.where(mask, d, jnp.float32(0.0))
            den_v[pl.ds(s, _L)] = jnp.maximum(d, 0.0)
            return 0

        lax.fori_loop(0, _GP, pass2, 0)

        pltpu.sync_copy(rgb_v, rgb_hbm.at[pl.ds(4 * base, 4 * _C)])
        pltpu.sync_copy(den_v, den_hbm.at[pl.ds(base, _C)])
        return 0

    lax.fori_loop(0, _NCHUNK, chunk_body, 0)


def kernel(xyz, grid):
    grid_lin = grid.transpose(0, 1, 3, 2).reshape(-1)
    xp = jnp.minimum(xyz.T.reshape(3 * _N), jnp.float32(1.0))
    rgb4, den, _ = _voxel_sc(xp, grid_lin)
    rgb = rgb4.reshape(_N // 128, 4, 128)[:, :3, :].transpose(0, 2, 1)
    rgb = jnp.minimum(rgb.reshape(_N, 3), jnp.float32(1.0))
    return rgb, den.reshape(_N, 1)


# speedup vs baseline: 1.1276x; 1.1276x over previous
import functools

import jax
import jax.numpy as jnp
from jax import lax
from jax.experimental import pallas as pl
from jax.experimental.pallas import tpu as pltpu
from jax.experimental.pallas import tpu_sc as plsc

_N = 1048576
_CELLS = 128
_NC, _NS, _L = 2, 16, 16
_NW = _NC * _NS
_PPW = _N // _NW
_C = 4096
_NCHUNK = _PPW // _C
_GP = _C // _L

_mesh = plsc.VectorSubcoreMesh(core_axis_name="c", subcore_axis_name="s")


def _bank_types():
    return [
        pltpu.VMEM((_C,), jnp.float32),
        pltpu.VMEM((_C,), jnp.int32),
        pltpu.VMEM((_C,), jnp.int32),
        pltpu.VMEM((_C,), jnp.int32),
        pltpu.VMEM((_C,), jnp.int32),
        pltpu.VMEM((_C,), jnp.int32),
        pltpu.VMEM((_C,), jnp.float32),
        pltpu.VMEM((_C,), jnp.float32),
        pltpu.VMEM((_C,), jnp.float32),
        pltpu.VMEM((_C,), jnp.float32),
        pltpu.SemaphoreType.DMA,
    ]


@functools.partial(
    pl.kernel,
    out_type=(
        jax.ShapeDtypeStruct((4 * _N,), jnp.float32),
        jax.ShapeDtypeStruct((_N,), jnp.float32),
    ),
    mesh=_mesh,
    compiler_params=pltpu.CompilerParams(
        needs_layout_passes=False, use_tc_tiling_on_sc=False),
    scratch_types=[
        pltpu.VMEM((_C,), jnp.float32),
        pltpu.VMEM((_C,), jnp.float32),
        pltpu.VMEM((_C,), jnp.float32),
        pltpu.VMEM((4 * _C,), jnp.float32),
        pltpu.VMEM((_C,), jnp.float32),
    ] + _bank_types() + _bank_types(),
)
def _voxel_sc(xp_hbm, grid_hbm, rgb_hbm, den_hbm,
              xv, yv, zv, rgb_v, den_v, *banks):
    wid = lax.axis_index("s") * _NC + lax.axis_index("c")
    bank_a = banks[:11]
    bank_b = banks[11:]

    def to_cell(v):
        i = (v * jnp.float32(_CELLS) + jnp.float32(_CELLS // 2)).astype(jnp.int32)
        return jnp.clip(i, 0, _CELLS - 1)

    zeros16 = jnp.zeros((_L,), jnp.int32)

    def zinit(i, _):
        bank_a[2][pl.ds(i * _L, _L)] = zeros16
        bank_b[2][pl.ds(i * _L, _L)] = zeros16
        return 0

    lax.fori_loop(0, _GP, zinit, 0)

    def pass1_fire(bank, ci):
        cond_v, pos_v = bank[0], bank[1]
        idx_refs = bank[2:6]
        sem = bank[10]
        base = wid * _PPW + ci * _C
        pltpu.sync_copy(xp_hbm.at[pl.ds(base, _C)], xv)
        pltpu.sync_copy(xp_hbm.at[pl.ds(_N + base, _C)], yv)
        pltpu.sync_copy(xp_hbm.at[pl.ds(2 * _N + base, _C)], zv)

        def pass1(g, off):
            s = g * _L
            x = xv[pl.ds(s, _L)]
            y = yv[pl.ds(s, _L)]
            z = zv[pl.ds(s, _L)]
            half = jnp.float32(0.5)
            cond = ((jnp.abs(x) < half) & (jnp.abs(y) < half)
                    & (jnp.abs(z) < half))
            maski = cond.astype(jnp.int32)
            e = (to_cell(x) * 128 + to_cell(y)) * 512 + to_cell(z)
            cond_v[pl.ds(s, _L)] = jnp.where(cond, 1.0, 0.0).astype(jnp.float32)
            pos_v[pl.ds(s, _L)] = off + plsc.cumsum(maski) - 1
            plsc.store_compressed(idx_refs[0].at[pl.ds(off, _L)], e, mask=cond)
            return off + jnp.sum(maski)

        cnt = lax.fori_loop(0, _GP, pass1, jnp.int32(0))
        ndma = (cnt + 127) >> 7

        def derive(g, _):
            s = g * _L
            b = idx_refs[0][pl.ds(s, _L)]
            idx_refs[1][pl.ds(s, _L)] = b + 128
            idx_refs[2][pl.ds(s, _L)] = b + 256
            idx_refs[3][pl.ds(s, _L)] = b + 384
            return 0

        lax.fori_loop(0, ndma * 8, derive, 0)

        def fire(j, _):
            for c in range(4):
                pltpu.make_async_copy(
                    grid_hbm.at[idx_refs[c].at[pl.ds(j * 128, 128)]],
                    bank[6 + c].at[pl.ds(j * 128, 128)], sem).start()
            return 0

        lax.fori_loop(0, ndma, fire, 0)
        return ndma

    def drain_pass2(bank, ci, ndma):
        cond_v, pos_v = bank[0], bank[1]
        idx_refs = bank[2:6]
        val_refs = bank[6:10]
        sem = bank[10]

        def drain(j, _):
            for c in range(4):
                pltpu.make_async_copy(
                    grid_hbm.at[idx_refs[c].at[pl.ds(j * 128, 128)]],
                    val_refs[c].at[pl.ds(j * 128, 128)], sem).wait()
            return 0

        lax.fori_loop(0, ndma, drain, 0)

        def pass2(g, _):
            s = g * _L
            condf = cond_v[pl.ds(s, _L)]
            mask = condf > jnp.float32(0.5)
            pos = pos_v[pl.ds(s, _L)]
            obase = (g >> 3) * 512 + (g & 7) * _L
            one = jnp.float32(1.0)
            for c in range(3):
                v = plsc.load_gather(val_refs[c], [pos], mask=mask)
                v = jnp.where(mask, v, jnp.float32(0.0))
                rgb_v[pl.ds(obase + c * 128, _L)] = one / (one + jnp.exp(-v))
            d = plsc.load_gather(val_refs[3], [pos], mask=mask)
            d = jnp.where(mask, d, jnp.float32(0.0))
            den_v[pl.ds(s, _L)] = jnp.maximum(d, 0.0)
            return 0

        lax.fori_loop(0, _GP, pass2, 0)
        base = wid * _PPW + ci * _C
        pltpu.sync_copy(rgb_v, rgb_hbm.at[pl.ds(4 * base, 4 * _C)])
        pltpu.sync_copy(den_v, den_hbm.at[pl.ds(base, _C)])

    def macro(m, ndma_b_prev):
        ndma_a = pass1_fire(bank_a, 2 * m)

        @pl.when(m > 0)
        def _():
            drain_pass2(bank_b, 2 * m - 1, ndma_b_prev)

        ndma_b = pass1_fire(bank_b, 2 * m + 1)
        drain_pass2(bank_a, 2 * m, ndma_a)
        return ndma_b

    ndma_last = lax.fori_loop(0, _NCHUNK // 2, macro, jnp.int32(0))
    drain_pass2(bank_b, _NCHUNK - 1, ndma_last)


def kernel(xyz, grid):
    grid_lin = grid.transpose(0, 1, 3, 2).reshape(-1)
    xp = jnp.minimum(xyz.T.reshape(3 * _N), jnp.float32(1.0))
    rgb4, den = _voxel_sc(xp, grid_lin)
    rgb = rgb4.reshape(_N // 128, 4, 128)[:, :3, :].transpose(0, 2, 1)
    rgb = jnp.minimum(rgb.reshape(_N, 3), jnp.float32(1.0))
    return rgb, den.reshape(_N, 1)


# speedup vs baseline: 1.1287x; 1.0009x over previous
import functools

import jax
import jax.numpy as jnp
from jax import lax
from jax.experimental import pallas as pl
from jax.experimental.pallas import tpu as pltpu
from jax.experimental.pallas import tpu_sc as plsc

_N = 1048576
_CELLS = 128
_NC, _NS, _L = 2, 16, 16
_NW = _NC * _NS
_PPW = _N // _NW
_C = 4096
_NCHUNK = _PPW // _C
_GP = _C // _L

_mesh = plsc.VectorSubcoreMesh(core_axis_name="c", subcore_axis_name="s")


def _bank_types():
    return [
        pltpu.VMEM((_C,), jnp.float32),
        pltpu.VMEM((_C,), jnp.int32),
        pltpu.VMEM((_C,), jnp.int32),
        pltpu.VMEM((_C,), jnp.int32),
        pltpu.VMEM((_C,), jnp.int32),
        pltpu.VMEM((_C,), jnp.int32),
        pltpu.VMEM((_C,), jnp.float32),
        pltpu.VMEM((_C,), jnp.float32),
        pltpu.VMEM((_C,), jnp.float32),
        pltpu.VMEM((_C,), jnp.float32),
        pltpu.SemaphoreType.DMA,
    ]


@functools.partial(
    pl.kernel,
    out_type=(
        jax.ShapeDtypeStruct((4 * _N,), jnp.float32),
        jax.ShapeDtypeStruct((_N,), jnp.float32),
    ),
    mesh=_mesh,
    compiler_params=pltpu.CompilerParams(
        needs_layout_passes=False, use_tc_tiling_on_sc=False),
    scratch_types=[
        pltpu.VMEM((_C,), jnp.float32),
        pltpu.VMEM((_C,), jnp.float32),
        pltpu.VMEM((_C,), jnp.float32),
        pltpu.VMEM((4 * _C,), jnp.float32),
        pltpu.VMEM((_C,), jnp.float32),
    ] + _bank_types() + _bank_types(),
)
def _voxel_sc(xp_hbm, grid_hbm, rgb_hbm, den_hbm,
              xv, yv, zv, rgb_v, den_v, *banks):
    wid = lax.axis_index("s") * _NC + lax.axis_index("c")
    bank_a = banks[:11]
    bank_b = banks[11:]

    def to_cell(v):
        i = (v * jnp.float32(_CELLS) + jnp.float32(_CELLS // 2)).astype(jnp.int32)
        return jnp.clip(i, 0, _CELLS - 1)

    zeros16 = jnp.zeros((_L,), jnp.int32)

    def zinit(i, _):
        bank_a[2][pl.ds(i * _L, _L)] = zeros16
        bank_b[2][pl.ds(i * _L, _L)] = zeros16
        return 0

    lax.fori_loop(0, _GP, zinit, 0)

    def pass1_fire(bank, ci):
        cond_v, pos_v = bank[0], bank[1]
        idx_refs = bank[2:6]
        sem = bank[10]
        base = wid * _PPW + ci * _C
        pltpu.sync_copy(xp_hbm.at[pl.ds(base, _C)], xv)
        pltpu.sync_copy(xp_hbm.at[pl.ds(_N + base, _C)], yv)
        pltpu.sync_copy(xp_hbm.at[pl.ds(2 * _N + base, _C)], zv)

        def pass1(g, off):
            for t in range(2):
                s = (g * 2 + t) * _L
                x = xv[pl.ds(s, _L)]
                y = yv[pl.ds(s, _L)]
                z = zv[pl.ds(s, _L)]
                half = jnp.float32(0.5)
                cond = ((jnp.abs(x) < half) & (jnp.abs(y) < half)
                        & (jnp.abs(z) < half))
                maski = cond.astype(jnp.int32)
                e = (to_cell(x) * 128 + to_cell(y)) * 512 + to_cell(z)
                cond_v[pl.ds(s, _L)] = cond.astype(jnp.float32)
                pos_v[pl.ds(s, _L)] = off + plsc.cumsum(maski) - 1
                plsc.store_compressed(idx_refs[0].at[pl.ds(off, _L)], e,
                                      mask=cond)
                off = off + plsc.all_reduce_population_count(cond)[0]
            return off

        cnt = lax.fori_loop(0, _GP // 2, pass1, jnp.int32(0))
        ndma = (cnt + 127) >> 7

        def derive(g, _):
            s = g * _L
            b = idx_refs[0][pl.ds(s, _L)]
            idx_refs[1][pl.ds(s, _L)] = b + 128
            idx_refs[2][pl.ds(s, _L)] = b + 256
            idx_refs[3][pl.ds(s, _L)] = b + 384
            return 0

        lax.fori_loop(0, ndma * 8, derive, 0)

        def fire(j, _):
            for c in range(4):
                pltpu.make_async_copy(
                    grid_hbm.at[idx_refs[c].at[pl.ds(j * 128, 128)]],
                    bank[6 + c].at[pl.ds(j * 128, 128)], sem).start()
            return 0

        lax.fori_loop(0, ndma, fire, 0)
        return ndma

    def drain_pass2(bank, ci, ndma):
        cond_v, pos_v = bank[0], bank[1]
        idx_refs = bank[2:6]
        val_refs = bank[6:10]
        sem = bank[10]

        def drain(j, _):
            for c in range(4):
                pltpu.make_async_copy(
                    grid_hbm.at[idx_refs[c].at[pl.ds(j * 128, 128)]],
                    val_refs[c].at[pl.ds(j * 128, 128)], sem).wait()
            return 0

        lax.fori_loop(0, ndma, drain, 0)

        def pass2(g, _):
            for t in range(2):
                g2 = g * 2 + t
                s = g2 * _L
                condf = cond_v[pl.ds(s, _L)]
                mask = condf > jnp.float32(0.5)
                pos = pos_v[pl.ds(s, _L)]
                obase = (g2 >> 3) * 512 + (g2 & 7) * _L
                one = jnp.float32(1.0)
                for c in range(3):
                    v = plsc.load_gather(val_refs[c], [pos], mask=mask)
                    v = jnp.where(mask, v, jnp.float32(0.0))
                    rgb_v[pl.ds(obase + c * 128, _L)] = one / (one + jnp.exp(-v))
                d = plsc.load_gather(val_refs[3], [pos], mask=mask)
                d = jnp.where(mask, d, jnp.float32(0.0))
                den_v[pl.ds(s, _L)] = jnp.maximum(d, 0.0)
            return 0

        lax.fori_loop(0, _GP // 2, pass2, 0)
        base = wid * _PPW + ci * _C
        pltpu.sync_copy(rgb_v, rgb_hbm.at[pl.ds(4 * base, 4 * _C)])
        pltpu.sync_copy(den_v, den_hbm.at[pl.ds(base, _C)])

    def macro(m, ndma_b_prev):
        ndma_a = pass1_fire(bank_a, 2 * m)

        @pl.when(m > 0)
        def _():
            drain_pass2(bank_b, 2 * m - 1, ndma_b_prev)

        ndma_b = pass1_fire(bank_b, 2 * m + 1)
        drain_pass2(bank_a, 2 * m, ndma_a)
        return ndma_b

    ndma_last = lax.fori_loop(0, _NCHUNK // 2, macro, jnp.int32(0))
    drain_pass2(bank_b, _NCHUNK - 1, ndma_last)


def kernel(xyz, grid):
    grid_lin = grid.transpose(0, 1, 3, 2).reshape(-1)
    xp = jnp.minimum(xyz.T.reshape(3 * _N), jnp.float32(1.0))
    rgb4, den = _voxel_sc(xp, grid_lin)
    rgb = rgb4.reshape(_N // 128, 4, 128)[:, :3, :].transpose(0, 2, 1)
    rgb = jnp.minimum(rgb.reshape(_N, 3), jnp.float32(1.0))
    return rgb, den.reshape(_N, 1)


# speedup vs baseline: 1.3484x; 1.1947x over previous
import functools

import jax
import jax.numpy as jnp
from jax import lax
from jax.experimental import pallas as pl
from jax.experimental.pallas import tpu as pltpu
from jax.experimental.pallas import tpu_sc as plsc

_N = 1048576
_CELLS = 128
_NC, _NS, _L = 2, 16, 16
_NW = _NC * _NS
_PPW = _N // _NW
_C = 4096
_NCHUNK = _PPW // _C
_GP = _C // _L

_mesh = plsc.VectorSubcoreMesh(core_axis_name="c", subcore_axis_name="s")


def _bank_types():
    return [
        pltpu.VMEM((_C,), jnp.float32),
        pltpu.VMEM((_C,), jnp.int32),
        pltpu.VMEM((_C,), jnp.int32),
        pltpu.VMEM((_C,), jnp.int32),
        pltpu.VMEM((_C,), jnp.int32),
        pltpu.VMEM((_C,), jnp.int32),
        pltpu.VMEM((_C,), jnp.float32),
        pltpu.VMEM((_C,), jnp.float32),
        pltpu.VMEM((_C,), jnp.float32),
        pltpu.VMEM((_C,), jnp.float32),
        pltpu.SemaphoreType.DMA,
    ]


@functools.partial(
    pl.kernel,
    out_type=(
        jax.ShapeDtypeStruct((4 * _N,), jnp.float32),
        jax.ShapeDtypeStruct((_N,), jnp.float32),
    ),
    mesh=_mesh,
    compiler_params=pltpu.CompilerParams(
        needs_layout_passes=False, use_tc_tiling_on_sc=False),
    scratch_types=[
        pltpu.VMEM((4 * _C,), jnp.float32),
        pltpu.VMEM((4 * _C,), jnp.float32),
        pltpu.VMEM((_C,), jnp.float32),
    ] + _bank_types() + _bank_types(),
)
def _voxel_sc(xp_hbm, grid_hbm, rgb_hbm, den_hbm,
              xyz_v, rgb_v, den_v, *banks):
    wid = lax.axis_index("s") * _NC + lax.axis_index("c")
    bank_a = banks[:11]
    bank_b = banks[11:]

    def to_cell(v):
        i = (v * jnp.float32(_CELLS) + jnp.float32(_CELLS // 2)).astype(jnp.int32)
        return jnp.clip(i, 0, _CELLS - 1)

    zeros16 = jnp.zeros((_L,), jnp.int32)

    def zinit(i, _):
        bank_a[2][pl.ds(i * _L, _L)] = zeros16
        bank_b[2][pl.ds(i * _L, _L)] = zeros16
        return 0

    lax.fori_loop(0, _GP, zinit, 0)

    def pass1_fire(bank, ci):
        cond_v, pos_v = bank[0], bank[1]
        idx_refs = bank[2:6]
        sem = bank[10]
        base = wid * _PPW + ci * _C
        pltpu.sync_copy(xp_hbm.at[pl.ds(4 * base, 4 * _C)], xyz_v)

        def pass1(g, off):
            for t in range(2):
                s = (g * 2 + t) * _L
                sb = ((s >> 7) << 9) + (s & 127)
                x = xyz_v[pl.ds(sb, _L)]
                y = xyz_v[pl.ds(sb + 128, _L)]
                z = xyz_v[pl.ds(sb + 256, _L)]
                half = jnp.float32(0.5)
                cond = ((jnp.abs(x) < half) & (jnp.abs(y) < half)
                        & (jnp.abs(z) < half))
                maski = cond.astype(jnp.int32)
                e = (to_cell(x) * 128 + to_cell(y)) * 512 + to_cell(z)
                cond_v[pl.ds(s, _L)] = cond.astype(jnp.float32)
                pos_v[pl.ds(s, _L)] = off + plsc.cumsum(maski) - 1
                plsc.store_compressed(idx_refs[0].at[pl.ds(off, _L)], e,
                                      mask=cond)
                off = off + plsc.all_reduce_population_count(cond)[0]
            return off

        cnt = lax.fori_loop(0, _GP // 2, pass1, jnp.int32(0))
        ndma = (cnt + 127) >> 7

        def derive(g, _):
            s = g * _L
            b = idx_refs[0][pl.ds(s, _L)]
            idx_refs[1][pl.ds(s, _L)] = b + 128
            idx_refs[2][pl.ds(s, _L)] = b + 256
            idx_refs[3][pl.ds(s, _L)] = b + 384
            return 0

        lax.fori_loop(0, ndma * 8, derive, 0)

        def fire(j, _):
            for c in range(4):
                pltpu.make_async_copy(
                    grid_hbm.at[idx_refs[c].at[pl.ds(j * 128, 128)]],
                    bank[6 + c].at[pl.ds(j * 128, 128)], sem).start()
            return 0

        lax.fori_loop(0, ndma, fire, 0)
        return ndma

    def drain_pass2(bank, ci, ndma):
        cond_v, pos_v = bank[0], bank[1]
        idx_refs = bank[2:6]
        val_refs = bank[6:10]
        sem = bank[10]

        def drain(j, _):
            for c in range(4):
                pltpu.make_async_copy(
                    grid_hbm.at[idx_refs[c].at[pl.ds(j * 128, 128)]],
                    val_refs[c].at[pl.ds(j * 128, 128)], sem).wait()
            return 0

        lax.fori_loop(0, ndma, drain, 0)

        def pass2(g, _):
            for t in range(2):
                g2 = g * 2 + t
                s = g2 * _L
                condf = cond_v[pl.ds(s, _L)]
                mask = condf > jnp.float32(0.5)
                pos = pos_v[pl.ds(s, _L)]
                obase = (g2 >> 3) * 512 + (g2 & 7) * _L
                one = jnp.float32(1.0)
                for c in range(3):
                    v = plsc.load_gather(val_refs[c], [pos], mask=mask)
                    v = jnp.where(mask, v, jnp.float32(0.0))
                    rgb_v[pl.ds(obase + c * 128, _L)] = one / (one + jnp.exp(-v))
                d = plsc.load_gather(val_refs[3], [pos], mask=mask)
                d = jnp.where(mask, d, jnp.float32(0.0))
                den_v[pl.ds(s, _L)] = jnp.maximum(d, 0.0)
            return 0

        lax.fori_loop(0, _GP // 2, pass2, 0)
        base = wid * _PPW + ci * _C
        pltpu.sync_copy(rgb_v, rgb_hbm.at[pl.ds(4 * base, 4 * _C)])
        pltpu.sync_copy(den_v, den_hbm.at[pl.ds(base, _C)])

    def macro(m, ndma_b_prev):
        ndma_a = pass1_fire(bank_a, 2 * m)

        @pl.when(m > 0)
        def _():
            drain_pass2(bank_b, 2 * m - 1, ndma_b_prev)

        ndma_b = pass1_fire(bank_b, 2 * m + 1)
        drain_pass2(bank_a, 2 * m, ndma_a)
        return ndma_b

    ndma_last = lax.fori_loop(0, _NCHUNK // 2, macro, jnp.int32(0))
    drain_pass2(bank_b, _NCHUNK - 1, ndma_last)


def kernel(xyz, grid):
    grid_lin = grid.transpose(0, 1, 3, 2).reshape(-1)
    xp = jnp.pad(xyz, ((0, 0), (0, 1)))
    xp = xp.reshape(_N // 128, 128, 4).transpose(0, 2, 1).reshape(4 * _N)
    xp = jnp.minimum(xp, jnp.float32(1.0))
    rgb4, den = _voxel_sc(xp, grid_lin)
    rgb = rgb4.reshape(_N // 128, 4, 128)[:, :3, :].transpose(0, 2, 1)
    rgb = jnp.minimum(rgb.reshape(_N, 3), jnp.float32(1.0))
    return rgb, den.reshape(_N, 1)
